# baseline (speedup 1.0000x reference)
import functools

import jax
import jax.numpy as jnp
from jax import lax
from jax.experimental import pallas as pl
from jax.experimental.pallas import tpu as pltpu
from jax.experimental.pallas import tpu_sc as plsc

_D = 64
_K = 1024
_TN = 3072


def _vq_body(xt_ref, cbt_ref, csq_ref, idx_ref, loss_ref):
    xt = xt_ref[...]
    cbt2 = -2.0 * cbt_ref[...]
    xsq = jnp.sum(xt * xt, axis=0, keepdims=True)
    s2 = lax.dot_general(cbt2, xt, (((0,), (0,)), ((), ())),
                         preferred_element_type=jnp.float32)
    dist = (xsq + s2) + csq_ref[...]
    m = jnp.min(dist, axis=0, keepdims=True)
    iota = lax.broadcasted_iota(jnp.int32, dist.shape, 0)
    idx_ref[...] = jnp.min(jnp.where(dist == m, iota, _K), axis=0)

    @pl.when(pl.program_id(0) == 0)
    def _():
        loss_ref[0, 0] = 0.0

    loss_ref[0, 0] += jnp.sum(m)


def _argmin_and_loss(xt, cbt, c_sq, start, steps):
    idx, loss = pl.pallas_call(
        _vq_body,
        grid=(steps,),
        in_specs=[
            pl.BlockSpec((_D, _TN), lambda i: (0, start + i)),
            pl.BlockSpec((_D, _K), lambda i: (0, 0)),
            pl.BlockSpec((_K, 1), lambda i: (0, 0)),
        ],
        out_specs=[
            pl.BlockSpec((_TN,), lambda i: (i,)),
            pl.BlockSpec((1, 1), lambda i: (0, 0), memory_space=pltpu.SMEM),
        ],
        out_shape=[
            jax.ShapeDtypeStruct((steps * _TN,), jnp.int32),
            jax.ShapeDtypeStruct((1, 1), jnp.float32),
        ],
    )(xt, cbt, c_sq)
    return idx, loss


def _make_sc_gather(n):
    info = plsc.get_sparse_core_info()
    nw = info.num_cores * info.num_subcores
    b_per_w = n // nw
    chunk = 96
    nch = b_per_w // chunk
    mesh = plsc.VectorSubcoreMesh(core_axis_name="c", subcore_axis_name="s")

    @functools.partial(
        pl.kernel, mesh=mesh,
        compiler_params=pltpu.CompilerParams(use_tc_tiling_on_sc=False),
        out_type=jax.ShapeDtypeStruct((n, _D), jnp.float32),
        scratch_types=[
            pltpu.VMEM((b_per_w,), jnp.int32),
            pltpu.VMEM((b_per_w, _D), jnp.float32),
            pltpu.SemaphoreType.DMA,
        ],
    )
    def gather_k(table_hbm, idx_hbm, out_hbm, idx_v, rows_v, sem):
        wid = lax.axis_index("s") * info.num_cores + lax.axis_index("c")
        base = wid * b_per_w
        pltpu.sync_copy(idx_hbm.at[pl.ds(base, b_per_w)], idx_v)
        copies = []
        for j in range(nch):
            copies.append(pltpu.async_copy(
                table_hbm.at[idx_v.at[pl.ds(j * chunk, chunk)]],
                rows_v.at[pl.ds(j * chunk, chunk)], sem))
        for c in copies:
            c.wait()
        pltpu.sync_copy(rows_v, out_hbm.at[pl.ds(base, b_per_w)])

    return gather_k


def kernel(inputs, codebook):
    n = inputs.shape[0]
    flat = inputs.reshape(-1, _D)
    half = n // 2
    steps = half // _TN
    xt = flat.T
    cbt = codebook.T
    c_sq = jnp.sum(codebook ** 2, axis=1)[:, None]
    gather = _make_sc_gather(half)
    idx1, ms1 = _argmin_and_loss(xt, cbt, c_sq, 0, steps)
    q1 = gather(codebook, idx1)
    idx2, ms2 = _argmin_and_loss(xt, cbt, c_sq, steps, steps)
    q2 = gather(codebook, idx2)
    mse = (ms1[0, 0] + ms2[0, 0]) / (n * _D)
    loss = mse + 0.25 * mse
    quantized = jnp.concatenate([q1, q2], axis=0)
    idx = jnp.concatenate([idx1, idx2])
    return loss, quantized, idx


# speedup vs baseline: 1.1029x; 1.1029x over previous
import functools

import jax
import jax.numpy as jnp
from jax import lax
from jax.experimental import pallas as pl
from jax.experimental.pallas import tpu as pltpu
from jax.experimental.pallas import tpu_sc as plsc

_D = 64
_K = 1024
_TN = 6144


def _vq_body(n, xt_ref, cbt_ref, csq_ref, idx_ref, loss_ref):
    xt = xt_ref[...]
    cbt2 = -2.0 * cbt_ref[...]
    xsq = jnp.sum(xt * xt, axis=0, keepdims=True)
    s2 = lax.dot_general(cbt2, xt, (((0,), (0,)), ((), ())),
                         preferred_element_type=jnp.float32)
    dist = (xsq + s2) + csq_ref[...]
    m = jnp.min(dist, axis=0, keepdims=True)
    iota = lax.broadcasted_iota(jnp.int32, dist.shape, 0)
    idx_ref[...] = jnp.min(jnp.where(dist == m, iota, _K), axis=0)

    @pl.when(pl.program_id(0) == 0)
    def _():
        loss_ref[0, 0] = 0.0

    loss_ref[0, 0] += jnp.sum(m)

    @pl.when(pl.program_id(0) == pl.num_programs(0) - 1)
    def _():
        mse = loss_ref[0, 0] / (n * _D)
        loss_ref[0, 0] = mse + 0.25 * mse


def _argmin_and_loss(flat, codebook):
    n = flat.shape[0]
    c_sq = jnp.sum(codebook ** 2, axis=1)[:, None]
    idx, loss = pl.pallas_call(
        functools.partial(_vq_body, n),
        grid=(n // _TN,),
        in_specs=[
            pl.BlockSpec((_D, _TN), lambda i: (0, i)),
            pl.BlockSpec((_D, _K), lambda i: (0, 0)),
            pl.BlockSpec((_K, 1), lambda i: (0, 0)),
        ],
        out_specs=[
            pl.BlockSpec((_TN,), lambda i: (i,)),
            pl.BlockSpec((1, 1), lambda i: (0, 0), memory_space=pltpu.SMEM),
        ],
        out_shape=[
            jax.ShapeDtypeStruct((n,), jnp.int32),
            jax.ShapeDtypeStruct((1, 1), jnp.float32),
        ],
    )(flat.T, codebook.T, c_sq)
    return idx, loss


def _make_sc_gather(n):
    info = plsc.get_sparse_core_info()
    nw = info.num_cores * info.num_subcores
    b_per_w = n // nw
    chunk = 96
    nch = b_per_w // chunk
    mesh = plsc.VectorSubcoreMesh(core_axis_name="c", subcore_axis_name="s")

    @functools.partial(
        pl.kernel, mesh=mesh,
        compiler_params=pltpu.CompilerParams(use_tc_tiling_on_sc=False),
        out_type=jax.ShapeDtypeStruct((n, _D), jnp.float32),
        scratch_types=[
            pltpu.VMEM((b_per_w,), jnp.int32),
            pltpu.VMEM((b_per_w, _D), jnp.float32),
            pltpu.SemaphoreType.DMA,
        ],
    )
    def gather_k(table_hbm, idx_hbm, out_hbm, idx_v, rows_v, sem):
        wid = lax.axis_index("s") * info.num_cores + lax.axis_index("c")
        base = wid * b_per_w
        pltpu.sync_copy(idx_hbm.at[pl.ds(base, b_per_w)], idx_v)
        copies = []
        for j in range(nch):
            copies.append(pltpu.async_copy(
                table_hbm.at[idx_v.at[pl.ds(j * chunk, chunk)]],
                rows_v.at[pl.ds(j * chunk, chunk)], sem))
        for c in copies:
            c.wait()
        pltpu.sync_copy(rows_v, out_hbm.at[pl.ds(base, b_per_w)])

    return gather_k


def kernel(inputs, codebook):
    n = inputs.shape[0]
    flat = inputs.reshape(-1, _D)
    idx, loss = _argmin_and_loss(flat, codebook)
    quantized = _make_sc_gather(n)(codebook, idx)
    return loss[0, 0], quantized, idx
